# baseline (speedup 1.0000x reference)
import functools

import jax
import jax.numpy as jnp
from jax import lax
from jax.experimental import pallas as pl
from jax.experimental.pallas import tpu as pltpu
from jax.experimental.pallas import tpu_sc as plsc

_NC = 2
_NS = 16
_NW = _NC * _NS
_LANES = 16


def _gather_norm_fn(b_per_w, table_hbm, idx_hbm, mean_hbm, norm_hbm, out_hbm,
                    idx_v, vals_v, mean_v, norm_v):
    wid = lax.axis_index("s") * _NC + lax.axis_index("c")
    base = wid * b_per_w
    pltpu.sync_copy(idx_hbm.at[pl.ds(base, b_per_w)], idx_v)
    pltpu.sync_copy(mean_hbm, mean_v)
    pltpu.sync_copy(norm_hbm, norm_v)
    pltpu.sync_copy(table_hbm.at[idx_v], vals_v)
    scale = 1.0 / norm_v[...]
    shift = mean_v[...] * scale

    @pl.loop(0, b_per_w, step=_LANES)
    def _(i):
        vals_v[pl.ds(i, _LANES)] = vals_v[pl.ds(i, _LANES)] * scale - shift

    pltpu.sync_copy(vals_v, out_hbm.at[pl.ds(base, b_per_w)])


def kernel(inputs, categ_bias, moving_mean, moving_norm):
    batch = inputs.shape[0]
    idx = inputs.reshape(batch).astype(jnp.int32)
    table = categ_bias.reshape(-1)
    mean16 = jnp.broadcast_to(moving_mean.astype(jnp.float32), (_LANES,))
    norm16 = jnp.broadcast_to(moving_norm.astype(jnp.float32), (_LANES,))
    b_per_w = batch // _NW

    mesh = plsc.VectorSubcoreMesh(core_axis_name="c", subcore_axis_name="s")
    run = pl.kernel(
        functools.partial(_gather_norm_fn, b_per_w),
        out_type=jax.ShapeDtypeStruct((batch,), jnp.float32),
        mesh=mesh,
        scratch_types=[
            pltpu.VMEM((b_per_w,), jnp.int32),
            pltpu.VMEM((b_per_w,), jnp.float32),
            pltpu.VMEM((_LANES,), jnp.float32),
            pltpu.VMEM((_LANES,), jnp.float32),
        ],
    )
    out = run(table, idx, mean16, norm16)
    return out.reshape(batch, 1)


# speedup vs baseline: 1.0932x; 1.0932x over previous
import functools

import jax
import jax.numpy as jnp
from jax import lax
from jax.experimental import pallas as pl
from jax.experimental.pallas import tpu as pltpu
from jax.experimental.pallas import tpu_sc as plsc

_NC = 2
_NS = 16
_NW = _NC * _NS
_LANES = 16


def _gather_norm_fn(b_per_w, table_hbm, idx_hbm, mn_hbm, out_hbm,
                    idx_v, vals_v, mn_v, sem_idx, sem_mn, sem_g):
    wid = lax.axis_index("s") * _NC + lax.axis_index("c")
    base = wid * b_per_w
    idx_cp = pltpu.async_copy(idx_hbm.at[pl.ds(base, b_per_w)], idx_v, sem_idx)
    mn_cp = pltpu.async_copy(mn_hbm, mn_v, sem_mn)
    idx_cp.wait()
    g_cp = pltpu.async_copy(table_hbm.at[idx_v], vals_v, sem_g)
    mn_cp.wait()
    scale = 1.0 / mn_v[pl.ds(_LANES, _LANES)]
    shift = mn_v[pl.ds(0, _LANES)] * scale
    g_cp.wait()

    @pl.loop(0, b_per_w, step=_LANES)
    def _(i):
        vals_v[pl.ds(i, _LANES)] = vals_v[pl.ds(i, _LANES)] * scale - shift

    pltpu.sync_copy(vals_v, out_hbm.at[pl.ds(base, b_per_w)])


def kernel(inputs, categ_bias, moving_mean, moving_norm):
    batch = inputs.shape[0]
    idx = inputs.reshape(batch).astype(jnp.int32)
    table = categ_bias.reshape(-1)
    mn = jnp.concatenate([
        jnp.broadcast_to(moving_mean.astype(jnp.float32), (_LANES,)),
        jnp.broadcast_to(moving_norm.astype(jnp.float32), (_LANES,)),
    ])
    b_per_w = batch // _NW

    mesh = plsc.VectorSubcoreMesh(core_axis_name="c", subcore_axis_name="s")
    run = pl.kernel(
        functools.partial(_gather_norm_fn, b_per_w),
        out_type=jax.ShapeDtypeStruct((batch,), jnp.float32),
        mesh=mesh,
        scratch_types=[
            pltpu.VMEM((b_per_w,), jnp.int32),
            pltpu.VMEM((b_per_w,), jnp.float32),
            pltpu.VMEM((2 * _LANES,), jnp.float32),
            pltpu.SemaphoreType.DMA,
            pltpu.SemaphoreType.DMA,
            pltpu.SemaphoreType.DMA,
        ],
    )
    out = run(table, idx, mn)
    return out.reshape(batch, 1)


# speedup vs baseline: 1.1337x; 1.0371x over previous
import functools

import jax
import jax.numpy as jnp
from jax import lax
from jax.experimental import pallas as pl
from jax.experimental.pallas import tpu as pltpu
from jax.experimental.pallas import tpu_sc as plsc

_NC = 2
_NS = 16
_NW = _NC * _NS
_LANES = 16


def _gather_norm_fn(b_per_w, table_hbm, idx_hbm, mean_hbm, norm_hbm, out_hbm,
                    idx_v, vals_v, zidx_v, mean_v, norm_v,
                    sem_idx, sem_m, sem_n, sem_g):
    wid = lax.axis_index("s") * _NC + lax.axis_index("c")
    base = wid * b_per_w
    zidx_v[...] = jnp.zeros((_LANES,), jnp.int32)
    idx_cp = pltpu.async_copy(idx_hbm.at[pl.ds(base, b_per_w)], idx_v, sem_idx)
    m_cp = pltpu.async_copy(mean_hbm.at[zidx_v], mean_v, sem_m)
    n_cp = pltpu.async_copy(norm_hbm.at[zidx_v], norm_v, sem_n)
    idx_cp.wait()
    g_cp = pltpu.async_copy(table_hbm.at[idx_v], vals_v, sem_g)
    m_cp.wait()
    n_cp.wait()
    scale = 1.0 / norm_v[...]
    shift = mean_v[...] * scale
    g_cp.wait()

    @pl.loop(0, b_per_w, step=_LANES)
    def _(i):
        vals_v[pl.ds(i, _LANES)] = vals_v[pl.ds(i, _LANES)] * scale - shift

    pltpu.sync_copy(vals_v, out_hbm.at[pl.ds(base, b_per_w)])


def kernel(inputs, categ_bias, moving_mean, moving_norm):
    batch = inputs.shape[0]
    idx = inputs.reshape(batch).astype(jnp.int32)
    table = categ_bias.reshape(-1)
    b_per_w = batch // _NW

    mesh = plsc.VectorSubcoreMesh(core_axis_name="c", subcore_axis_name="s")
    run = pl.kernel(
        functools.partial(_gather_norm_fn, b_per_w),
        out_type=jax.ShapeDtypeStruct((batch,), jnp.float32),
        mesh=mesh,
        scratch_types=[
            pltpu.VMEM((b_per_w,), jnp.int32),
            pltpu.VMEM((b_per_w,), jnp.float32),
            pltpu.VMEM((_LANES,), jnp.int32),
            pltpu.VMEM((_LANES,), jnp.float32),
            pltpu.VMEM((_LANES,), jnp.float32),
            pltpu.SemaphoreType.DMA,
            pltpu.SemaphoreType.DMA,
            pltpu.SemaphoreType.DMA,
            pltpu.SemaphoreType.DMA,
        ],
    )
    out = run(table, idx, moving_mean.astype(jnp.float32),
              moving_norm.astype(jnp.float32))
    return out.reshape(batch, 1)


# speedup vs baseline: 1.1387x; 1.0044x over previous
import functools

import jax
import jax.numpy as jnp
from jax import lax
from jax.experimental import pallas as pl
from jax.experimental.pallas import tpu as pltpu
from jax.experimental.pallas import tpu_sc as plsc

_NC = 2
_NS = 16
_NW = _NC * _NS
_LANES = 16


def _gather_norm_fn(b_per_w, table_hbm, idx_hbm, mean_hbm, norm_hbm, out_hbm,
                    idx_v, vals_v, zidx_v, mean_v, norm_v,
                    sem_i0, sem_i1, sem_m, sem_n, sem_g0, sem_g1,
                    sem_o0, sem_o1):
    wid = lax.axis_index("s") * _NC + lax.axis_index("c")
    base = wid * b_per_w
    half = b_per_w // 2
    i_cp0 = pltpu.async_copy(idx_hbm.at[pl.ds(base, half)],
                             idx_v.at[pl.ds(0, half)], sem_i0)
    i_cp1 = pltpu.async_copy(idx_hbm.at[pl.ds(base + half, half)],
                             idx_v.at[pl.ds(half, half)], sem_i1)
    zidx_v[...] = jnp.zeros((_LANES,), jnp.int32)
    m_cp = pltpu.async_copy(mean_hbm.at[zidx_v], mean_v, sem_m)
    n_cp = pltpu.async_copy(norm_hbm.at[zidx_v], norm_v, sem_n)
    i_cp0.wait()
    g_cp0 = pltpu.async_copy(table_hbm.at[idx_v.at[pl.ds(0, half)]],
                             vals_v.at[pl.ds(0, half)], sem_g0)
    i_cp1.wait()
    g_cp1 = pltpu.async_copy(table_hbm.at[idx_v.at[pl.ds(half, half)]],
                             vals_v.at[pl.ds(half, half)], sem_g1)
    m_cp.wait()
    n_cp.wait()
    scale = 1.0 / norm_v[...]
    shift = mean_v[...] * scale
    g_cp0.wait()

    @pl.loop(0, half, step=_LANES)
    def _(i):
        vals_v[pl.ds(i, _LANES)] = vals_v[pl.ds(i, _LANES)] * scale - shift

    o_cp0 = pltpu.async_copy(vals_v.at[pl.ds(0, half)],
                             out_hbm.at[pl.ds(base, half)], sem_o0)
    g_cp1.wait()

    @pl.loop(half, b_per_w, step=_LANES)
    def _(i):
        vals_v[pl.ds(i, _LANES)] = vals_v[pl.ds(i, _LANES)] * scale - shift

    o_cp1 = pltpu.async_copy(vals_v.at[pl.ds(half, half)],
                             out_hbm.at[pl.ds(base + half, half)], sem_o1)
    o_cp0.wait()
    o_cp1.wait()


def kernel(inputs, categ_bias, moving_mean, moving_norm):
    batch = inputs.shape[0]
    idx = inputs.reshape(batch).astype(jnp.int32)
    table = categ_bias.reshape(-1)
    b_per_w = batch // _NW

    mesh = plsc.VectorSubcoreMesh(core_axis_name="c", subcore_axis_name="s")
    run = pl.kernel(
        functools.partial(_gather_norm_fn, b_per_w),
        out_type=jax.ShapeDtypeStruct((batch,), jnp.float32),
        mesh=mesh,
        scratch_types=[
            pltpu.VMEM((b_per_w,), jnp.int32),
            pltpu.VMEM((b_per_w,), jnp.float32),
            pltpu.VMEM((_LANES,), jnp.int32),
            pltpu.VMEM((_LANES,), jnp.float32),
            pltpu.VMEM((_LANES,), jnp.float32),
            pltpu.SemaphoreType.DMA,
            pltpu.SemaphoreType.DMA,
            pltpu.SemaphoreType.DMA,
            pltpu.SemaphoreType.DMA,
            pltpu.SemaphoreType.DMA,
            pltpu.SemaphoreType.DMA,
            pltpu.SemaphoreType.DMA,
            pltpu.SemaphoreType.DMA,
        ],
    )
    out = run(table, idx, moving_mean.astype(jnp.float32),
              moving_norm.astype(jnp.float32))
    return out.reshape(batch, 1)
